# baseline (speedup 1.0000x reference)
import functools

import jax
import jax.numpy as jnp
import numpy as np
from jax.experimental import pallas as pl
from jax.experimental.pallas import tpu as pltpu
from jax.experimental.pallas import tpu_sc as plsc

EMBED_DIM = 36
OUT_DIMS = [72, 144, 288, 576]
GROUP_NUMS = [1024, 512, 256, 128]
K_NEIGHBORS = 40
ALPHA = 1000.0
BETA = 100.0

GT = 32


def _index_points(points, idx):
    B = points.shape[0]
    bidx = jnp.arange(B).reshape((B,) + (1,) * (idx.ndim - 1))
    return points[bidx, idx]


def _square_distance(src, dst):
    dist = -2.0 * jnp.matmul(src, jnp.transpose(dst, (0, 2, 1)))
    dist = dist + jnp.sum(src ** 2, -1)[:, :, None]
    dist = dist + jnp.sum(dst ** 2, -1)[:, None, :]
    return dist


_SC_CHUNK = 128


def _sc_gather(table, idx):
    V, Dp = table.shape
    M = idx.shape[0]
    NC, NS = 2, 16
    NW = NC * NS
    mpw = M // NW
    nch = mpw // _SC_CHUNK
    assert mpw % _SC_CHUNK == 0 and nch % 2 == 0

    mesh = plsc.VectorSubcoreMesh(core_axis_name="c", subcore_axis_name="s")

    @functools.partial(
        pl.kernel, mesh=mesh,
        out_type=jax.ShapeDtypeStruct((M, Dp), jnp.float32),
        scratch_types=[
            pltpu.VMEM((mpw,), jnp.int32),
            pltpu.VMEM((2, _SC_CHUNK, Dp), jnp.float32),
            pltpu.SemaphoreType.DMA,
        ],
        compiler_params=pltpu.CompilerParams(use_tc_tiling_on_sc=False),
    )
    def gather_kernel(table_hbm, idx_hbm, out_hbm, idx_v, rows_v, sem):
        wid = jax.lax.axis_index("s") * NC + jax.lax.axis_index("c")
        base0 = wid * mpw
        pltpu.sync_copy(idx_hbm.at[pl.ds(base0, mpw)], idx_v)
        pltpu.async_copy(
            table_hbm.at[idx_v.at[pl.ds(0, _SC_CHUNK)]], rows_v.at[0], sem)

        def body(j2, carry):
            for bf in range(2):
                j = 2 * j2 + bf
                pltpu.make_async_copy(
                    out_hbm.at[pl.ds(base0, _SC_CHUNK)],
                    rows_v.at[bf], sem).wait()

                @pl.when(j + 1 < nch)
                def _():
                    pltpu.async_copy(
                        table_hbm.at[idx_v.at[pl.ds((j + 1) * _SC_CHUNK,
                                                    _SC_CHUNK)]],
                        rows_v.at[1 - bf], sem)

                pltpu.sync_copy(
                    rows_v.at[bf],
                    out_hbm.at[pl.ds(base0 + j * _SC_CHUNK, _SC_CHUNK)])
            return carry

        jax.lax.fori_loop(0, nch // 2, body, 0)

    return gather_kernel(table, idx)


def _pass_b_body(T, C_in, Dp, rows_ref, xn_ref, w1t_ref, sf_ref, ph_ref,
                 y1_ref, h_ref, st_ref):
    b, t = pl.program_id(0), pl.program_id(1)

    arg = jax.lax.dot_general(
        xn_ref[...], sf_ref[...], (((1,), (0,)), ((), ())),
        precision=jax.lax.Precision.HIGHEST,
        preferred_element_type=jnp.float32)
    pe = jnp.sin(arg + ph_ref[...])

    rows = rows_ref[...]
    h = jnp.concatenate(
        [rows[:, 0:C_in], rows[:, Dp:Dp + C_in]], axis=1) + pe
    h_ref[...] = h

    y1 = jax.lax.dot_general(
        h, w1t_ref[...], (((1,), (0,)), ((), ())),
        precision=jax.lax.Precision.HIGHEST,
        preferred_element_type=jnp.float32)
    y1_ref[...] = y1

    @pl.when(t == 0)
    def _():
        st_ref[...] = jnp.zeros_like(st_ref)

    st_ref[0:1, :] += jnp.sum(y1, axis=0, keepdims=True)
    st_ref[1:2, :] += jnp.sum(y1 * y1, axis=0, keepdims=True)


def _pass_b(pair_rows, C_in, xn_rows, w1t, sf, phase):
    R_tot, Dp2 = pair_rows.shape
    Dp = Dp2 // 2
    C_out, hd = w1t.shape
    B = 8
    T = R_tot // (B * GT * K_NEIGHBORS)
    R = GT * K_NEIGHBORS
    return pl.pallas_call(
        functools.partial(_pass_b_body, T, C_in, Dp),
        grid=(B, T),
        in_specs=[
            pl.BlockSpec((R, Dp2), lambda b, t: (b * T + t, 0)),
            pl.BlockSpec((R, 8), lambda b, t: (b * T + t, 0)),
            pl.BlockSpec((C_out, hd), lambda b, t: (0, 0)),
            pl.BlockSpec((8, C_out), lambda b, t: (0, 0)),
            pl.BlockSpec((1, C_out), lambda b, t: (0, 0)),
        ],
        out_specs=[
            pl.BlockSpec((R, hd), lambda b, t: (b * T + t, 0)),
            pl.BlockSpec((R, C_out), lambda b, t: (b * T + t, 0)),
            pl.BlockSpec((8, hd), lambda b, t: (b, 0)),
        ],
        out_shape=[
            jax.ShapeDtypeStruct((R_tot, hd), jnp.float32),
            jax.ShapeDtypeStruct((R_tot, C_out), jnp.float32),
            jax.ShapeDtypeStruct((8 * B, hd), jnp.float32),
        ],
        compiler_params=pltpu.CompilerParams(
            dimension_semantics=("parallel", "arbitrary")),
    )(pair_rows, xn_rows, w1t, sf, phase)


def _pass_c_body(a1_ref, c1_ref, y1_ref, w2t_ref, y2_ref, st_ref):
    b, t = pl.program_id(0), pl.program_id(1)
    r1 = jnp.maximum(y1_ref[...] * a1_ref[...] + c1_ref[...], 0.0)
    y2 = jax.lax.dot_general(
        r1, w2t_ref[...], (((1,), (0,)), ((), ())),
        precision=jax.lax.Precision.HIGHEST,
        preferred_element_type=jnp.float32)
    y2_ref[...] = y2

    @pl.when(t == 0)
    def _():
        st_ref[...] = jnp.zeros_like(st_ref)

    st_ref[0:1, :] += jnp.sum(y2, axis=0, keepdims=True)
    st_ref[1:2, :] += jnp.sum(y2 * y2, axis=0, keepdims=True)


def _pass_c(y1_rows, w2t, a1, c1):
    R_tot, hd = y1_rows.shape
    C_out = w2t.shape[1]
    B = 8
    R = GT * K_NEIGHBORS
    T = R_tot // (B * R)
    return pl.pallas_call(
        _pass_c_body,
        grid=(B, T),
        in_specs=[
            pl.BlockSpec((1, hd), lambda b, t: (0, 0)),
            pl.BlockSpec((1, hd), lambda b, t: (0, 0)),
            pl.BlockSpec((R, hd), lambda b, t: (b * T + t, 0)),
            pl.BlockSpec((hd, C_out), lambda b, t: (0, 0)),
        ],
        out_specs=[
            pl.BlockSpec((R, C_out), lambda b, t: (b * T + t, 0)),
            pl.BlockSpec((8, C_out), lambda b, t: (b, 0)),
        ],
        out_shape=[
            jax.ShapeDtypeStruct((R_tot, C_out), jnp.float32),
            jax.ShapeDtypeStruct((8 * B, C_out), jnp.float32),
        ],
        compiler_params=pltpu.CompilerParams(
            dimension_semantics=("parallel", "arbitrary")),
    )(a1, c1, y1_rows, w2t)


def _pass_d_body(a2_ref, c2_ref, y2_ref, h_ref, o_ref):
    C_out = y2_ref.shape[1]
    hout = jnp.maximum(y2_ref[...] * a2_ref[...] + c2_ref[...] + h_ref[...], 0.0)
    hr = hout.reshape(GT, K_NEIGHBORS, C_out)
    o_ref[...] = jnp.max(hr, axis=1)


def _pass_d(y2_rows, h_rows, a2, c2):
    R_tot, C_out = y2_rows.shape
    B = 8
    R = GT * K_NEIGHBORS
    T = R_tot // (B * R)
    return pl.pallas_call(
        _pass_d_body,
        grid=(B, T),
        in_specs=[
            pl.BlockSpec((1, C_out), lambda b, t: (0, 0)),
            pl.BlockSpec((1, C_out), lambda b, t: (0, 0)),
            pl.BlockSpec((R, C_out), lambda b, t: (b * T + t, 0)),
            pl.BlockSpec((R, C_out), lambda b, t: (b * T + t, 0)),
        ],
        out_specs=pl.BlockSpec((GT, C_out), lambda b, t: (b * T + t, 0)),
        out_shape=jax.ShapeDtypeStruct((R_tot // K_NEIGHBORS, C_out), jnp.float32),
        compiler_params=pltpu.CompilerParams(
            dimension_semantics=("parallel", "parallel")),
    )(a2, c2, y2_rows, h_rows)


def _pe_consts(C_out):
    fd = C_out // 6
    freq = BETA / np.power(ALPHA, np.arange(fd, dtype=np.float64) / fd)
    sf = np.zeros((8, C_out), np.float32)
    phase = np.zeros((1, C_out), np.float32)
    for c in range(C_out):
        d = c // (2 * fd)
        t = c % (2 * fd)
        f = t if t < fd else t - fd
        sf[d, c] = freq[f]
        phase[0, c] = 0.0 if t < fd else np.pi / 2.0
    return jnp.asarray(sf), jnp.asarray(phase)


def _bn_affine(stats, gamma, beta, count):
    st = jnp.sum(stats.reshape(-1, 8, stats.shape[1]), axis=0)
    s, ss = st[0], st[1]
    mean = s / count
    var = ss / count - mean * mean
    a = gamma / jnp.sqrt(var + 1e-5)
    c = beta - mean * a
    return a[None, :], c[None, :]


def kernel(xyz, x, params):
    B, N = xyz.shape[0], xyz.shape[1]
    K = K_NEIGHBORS

    feat = jnp.einsum('oc,bcn->bon', params['w0'], x)
    m = jnp.mean(feat, axis=(0, 2), keepdims=True)
    v = jnp.var(feat, axis=(0, 2), keepdims=True)
    feat = (feat - m) / jnp.sqrt(v + 1e-5)
    feat = jax.nn.relu(feat * params['g0'][None, :, None]
                       + params['b0'][None, :, None])

    cur_xyz = xyz
    cur_rows = jnp.transpose(feat, (0, 2, 1))
    idx_key = jax.random.key(42)

    for i in range(4):
        Ni = cur_xyz.shape[1]
        G, C_out = GROUP_NUMS[i], OUT_DIMS[i]
        C_in = C_out // 2
        hd = C_out // 2

        fps_idx = jax.random.randint(
            jax.random.fold_in(idx_key, i), (B, G), 0, Ni)
        lc_xyz = _index_points(cur_xyz, fps_idx)

        dist = _square_distance(lc_xyz, cur_xyz)
        _, knn_idx = jax.lax.top_k(-dist, K)

        knn_xyz = _index_points(cur_xyz, knn_idx)

        Dp = (C_in + 15) // 16 * 16
        tbl = cur_rows.reshape(B * Ni, C_in)
        if Dp != C_in:
            tbl = jnp.pad(tbl, ((0, 0), (0, Dp - C_in)))
        base = jnp.arange(B, dtype=jnp.int32) * Ni
        gidx = knn_idx + base[:, None, None]
        lidx = jnp.broadcast_to(
            (fps_idx + base[:, None])[:, :, None], gidx.shape)
        idx_m = jnp.stack([gidx, lidx], axis=-1).reshape(-1)
        pair_rows = _sc_gather(tbl, idx_m).reshape(B * G * K, 2 * Dp)

        diff = knn_xyz - lc_xyz[:, :, None, :]
        std = jnp.std(diff, ddof=1)
        xn = diff / (std + 1e-5)
        xn_rows = jnp.concatenate(
            [xn, jnp.zeros(xn.shape[:3] + (5,), jnp.float32)], axis=-1)
        xn_rows = xn_rows.reshape(B * G * K, 8)

        sf, phase = _pe_consts(C_out)
        w1t = params['w1_%d' % i].T
        w2t = params['w2_%d' % i].T

        y1_rows, h_rows, st1 = _pass_b(
            pair_rows, C_in, xn_rows, w1t, sf, phase)

        cnt = float(B * G * K)
        a1, c1 = _bn_affine(st1, params['g1_%d' % i], params['be1_%d' % i], cnt)
        y2_rows, st2 = _pass_c(y1_rows, w2t, a1, c1)
        a2, c2 = _bn_affine(st2, params['g2_%d' % i], params['be2_%d' % i], cnt)
        new_rows = _pass_d(y2_rows, h_rows, a2, c2)

        cur_rows = new_rows.reshape(B, G, C_out)
        cur_xyz = lc_xyz

    return cur_xyz, jnp.transpose(cur_rows, (0, 2, 1))


# speedup vs baseline: 2.8923x; 2.8923x over previous
import functools

import jax
import jax.numpy as jnp
import numpy as np
from jax.experimental import pallas as pl
from jax.experimental.pallas import tpu as pltpu
from jax.experimental.pallas import tpu_sc as plsc

EMBED_DIM = 36
OUT_DIMS = [72, 144, 288, 576]
GROUP_NUMS = [1024, 512, 256, 128]
K_NEIGHBORS = 40
ALPHA = 1000.0
BETA = 100.0

GT = 32


def _index_points(points, idx):
    B = points.shape[0]
    bidx = jnp.arange(B).reshape((B,) + (1,) * (idx.ndim - 1))
    return points[bidx, idx]


def _square_distance(src, dst):
    dist = -2.0 * jnp.matmul(src, jnp.transpose(dst, (0, 2, 1)))
    dist = dist + jnp.sum(src ** 2, -1)[:, :, None]
    dist = dist + jnp.sum(dst ** 2, -1)[:, None, :]
    return dist


_SC_CHUNK = 128


def _sc_gather(table, idx):
    V, Dp = table.shape
    M = idx.shape[0]
    NC, NS = 2, 16
    NW = NC * NS
    mpw = M // NW
    nch = mpw // _SC_CHUNK
    assert mpw % _SC_CHUNK == 0 and nch % 2 == 0

    mesh = plsc.VectorSubcoreMesh(core_axis_name="c", subcore_axis_name="s")

    @functools.partial(
        pl.kernel, mesh=mesh,
        out_type=jax.ShapeDtypeStruct((M, Dp), jnp.float32),
        scratch_types=[
            pltpu.VMEM((mpw,), jnp.int32),
            pltpu.VMEM((2, _SC_CHUNK, Dp), jnp.float32),
            pltpu.SemaphoreType.DMA,
        ],
        compiler_params=pltpu.CompilerParams(use_tc_tiling_on_sc=False),
    )
    def gather_kernel(table_hbm, idx_hbm, out_hbm, idx_v, rows_v, sem):
        wid = jax.lax.axis_index("s") * NC + jax.lax.axis_index("c")
        base0 = wid * mpw
        pltpu.sync_copy(idx_hbm.at[pl.ds(base0, mpw)], idx_v)
        pltpu.async_copy(
            table_hbm.at[idx_v.at[pl.ds(0, _SC_CHUNK)]], rows_v.at[0], sem)

        def body(j2, carry):
            for bf in range(2):
                j = 2 * j2 + bf
                pltpu.make_async_copy(
                    out_hbm.at[pl.ds(base0, _SC_CHUNK)],
                    rows_v.at[bf], sem).wait()

                @pl.when(j + 1 < nch)
                def _():
                    pltpu.async_copy(
                        table_hbm.at[idx_v.at[pl.ds((j + 1) * _SC_CHUNK,
                                                    _SC_CHUNK)]],
                        rows_v.at[1 - bf], sem)

                pltpu.sync_copy(
                    rows_v.at[bf],
                    out_hbm.at[pl.ds(base0 + j * _SC_CHUNK, _SC_CHUNK)])
            return carry

        jax.lax.fori_loop(0, nch // 2, body, 0)

    return gather_kernel(table, idx)


GTK = 128


def _topk_body(xyzw_ref, d_ref, idx_ref, st_ref):
    Gt, N = d_ref.shape
    K = K_NEIGHBORS
    d = d_ref[...]
    iota = jax.lax.broadcasted_iota(jnp.int32, (Gt, N), 1).astype(jnp.float32)
    kiota = jax.lax.broadcasted_iota(jnp.int32, (Gt, K), 1).astype(jnp.float32)
    W = jnp.zeros((Gt, N), jnp.float32)
    idxs = jnp.zeros((Gt, K), jnp.float32)
    sd = jnp.zeros((Gt, 1), jnp.float32)
    big = jnp.float32(1e9)
    for k in range(K):
        m = jnp.min(d, axis=1, keepdims=True)
        idx = jnp.min(jnp.where(d == m, iota, big), axis=1, keepdims=True)
        sel = iota == idx
        W = W + sel.astype(jnp.float32)
        d = jnp.where(sel, jnp.float32(jnp.inf), d)
        sd = sd + m
        idxs = jnp.where(kiota == jnp.float32(k), idx, idxs)
    idx_ref[...] = idxs.astype(jnp.int32)
    p = jax.lax.dot_general(
        W, xyzw_ref[0], (((1,), (0,)), ((), ())),
        precision=jax.lax.Precision.HIGHEST,
        preferred_element_type=jnp.float32)
    row = jnp.sum(p, axis=0, keepdims=True)
    sdt = jnp.sum(sd, axis=0, keepdims=True)
    slot3 = (jax.lax.broadcasted_iota(jnp.int32, (1, 8), 1) == 3
             ).astype(jnp.float32)
    st_ref[...] = (row + sdt * slot3).reshape(1, 1, 8)


def _topk_select(dist, xyzw):
    B, G, N = dist.shape
    T = G // GTK
    idx, st = pl.pallas_call(
        _topk_body,
        grid=(B, T),
        in_specs=[
            pl.BlockSpec((1, N, 8), lambda b, t: (b, 0, 0)),
            pl.BlockSpec((GTK, N), lambda b, t: (b * T + t, 0)),
        ],
        out_specs=[
            pl.BlockSpec((GTK, K_NEIGHBORS), lambda b, t: (b * T + t, 0)),
            pl.BlockSpec((1, 1, 8), lambda b, t: (b * T + t, 0, 0)),
        ],
        out_shape=[
            jax.ShapeDtypeStruct((B * G, K_NEIGHBORS), jnp.int32),
            jax.ShapeDtypeStruct((B * T, 1, 8), jnp.float32),
        ],
        compiler_params=pltpu.CompilerParams(
            dimension_semantics=("parallel", "parallel")),
    )(xyzw, dist.reshape(B * G, N))
    return idx.reshape(B, G, K_NEIGHBORS), st


def _pass_b_body(T, C_in, Dp, rows_ref, lc_ref, inv_ref, w1t_ref, sf_ref,
                 ph_ref, y1_ref, h_ref, st_ref):
    b, t = pl.program_id(0), pl.program_id(1)
    K = K_NEIGHBORS

    rows = rows_ref[...]
    lcb = jnp.broadcast_to(
        lc_ref[...][:, None, :], (GT, K, Dp)).reshape(GT * K, Dp)
    xn = (rows[:, C_in:C_in + 8] - lcb[:, C_in:C_in + 8]) * inv_ref[0:1, 0:1]
    arg = jax.lax.dot_general(
        xn, sf_ref[...], (((1,), (0,)), ((), ())),
        precision=jax.lax.Precision.HIGHEST,
        preferred_element_type=jnp.float32)
    pe = jnp.sin(arg + ph_ref[...])
    h = jnp.concatenate(
        [rows[:, 0:C_in], lcb[:, 0:C_in]], axis=1) + pe
    h_ref[...] = h

    y1 = jax.lax.dot_general(
        h, w1t_ref[...], (((1,), (0,)), ((), ())),
        precision=jax.lax.Precision.HIGHEST,
        preferred_element_type=jnp.float32)
    y1_ref[...] = y1

    @pl.when(t == 0)
    def _():
        st_ref[...] = jnp.zeros_like(st_ref)

    st_ref[0:1, :] += jnp.sum(y1, axis=0, keepdims=True)
    st_ref[1:2, :] += jnp.sum(y1 * y1, axis=0, keepdims=True)


def _pass_b(knn_rows, lc_rows, C_in, invstd, w1t, sf, phase):
    R_tot, Dp = knn_rows.shape
    C_out, hd = w1t.shape
    B = 8
    T = R_tot // (B * GT * K_NEIGHBORS)
    R = GT * K_NEIGHBORS
    return pl.pallas_call(
        functools.partial(_pass_b_body, T, C_in, Dp),
        grid=(B, T),
        in_specs=[
            pl.BlockSpec((R, Dp), lambda b, t: (b * T + t, 0)),
            pl.BlockSpec((GT, Dp), lambda b, t: (b * T + t, 0)),
            pl.BlockSpec((1, 8), lambda b, t: (0, 0)),
            pl.BlockSpec((C_out, hd), lambda b, t: (0, 0)),
            pl.BlockSpec((8, C_out), lambda b, t: (0, 0)),
            pl.BlockSpec((1, C_out), lambda b, t: (0, 0)),
        ],
        out_specs=[
            pl.BlockSpec((R, hd), lambda b, t: (b * T + t, 0)),
            pl.BlockSpec((R, C_out), lambda b, t: (b * T + t, 0)),
            pl.BlockSpec((8, hd), lambda b, t: (b, 0)),
        ],
        out_shape=[
            jax.ShapeDtypeStruct((R_tot, hd), jnp.float32),
            jax.ShapeDtypeStruct((R_tot, C_out), jnp.float32),
            jax.ShapeDtypeStruct((8 * B, hd), jnp.float32),
        ],
        compiler_params=pltpu.CompilerParams(
            dimension_semantics=("parallel", "arbitrary")),
    )(knn_rows, lc_rows, invstd, w1t, sf, phase)


def _pass_c_body(a1_ref, c1_ref, y1_ref, w2t_ref, y2_ref, st_ref):
    b, t = pl.program_id(0), pl.program_id(1)
    r1 = jnp.maximum(y1_ref[...] * a1_ref[...] + c1_ref[...], 0.0)
    y2 = jax.lax.dot_general(
        r1, w2t_ref[...], (((1,), (0,)), ((), ())),
        precision=jax.lax.Precision.HIGHEST,
        preferred_element_type=jnp.float32)
    y2_ref[...] = y2

    @pl.when(t == 0)
    def _():
        st_ref[...] = jnp.zeros_like(st_ref)

    st_ref[0:1, :] += jnp.sum(y2, axis=0, keepdims=True)
    st_ref[1:2, :] += jnp.sum(y2 * y2, axis=0, keepdims=True)


def _pass_c(y1_rows, w2t, a1, c1):
    R_tot, hd = y1_rows.shape
    C_out = w2t.shape[1]
    B = 8
    R = GT * K_NEIGHBORS
    T = R_tot // (B * R)
    return pl.pallas_call(
        _pass_c_body,
        grid=(B, T),
        in_specs=[
            pl.BlockSpec((1, hd), lambda b, t: (0, 0)),
            pl.BlockSpec((1, hd), lambda b, t: (0, 0)),
            pl.BlockSpec((R, hd), lambda b, t: (b * T + t, 0)),
            pl.BlockSpec((hd, C_out), lambda b, t: (0, 0)),
        ],
        out_specs=[
            pl.BlockSpec((R, C_out), lambda b, t: (b * T + t, 0)),
            pl.BlockSpec((8, C_out), lambda b, t: (b, 0)),
        ],
        out_shape=[
            jax.ShapeDtypeStruct((R_tot, C_out), jnp.float32),
            jax.ShapeDtypeStruct((8 * B, C_out), jnp.float32),
        ],
        compiler_params=pltpu.CompilerParams(
            dimension_semantics=("parallel", "arbitrary")),
    )(a1, c1, y1_rows, w2t)


def _pass_d_body(a2_ref, c2_ref, y2_ref, h_ref, o_ref):
    C_out = y2_ref.shape[1]
    hout = jnp.maximum(y2_ref[...] * a2_ref[...] + c2_ref[...] + h_ref[...], 0.0)
    hr = hout.reshape(GT, K_NEIGHBORS, C_out)
    o_ref[...] = jnp.max(hr, axis=1)


def _pass_d(y2_rows, h_rows, a2, c2):
    R_tot, C_out = y2_rows.shape
    B = 8
    R = GT * K_NEIGHBORS
    T = R_tot // (B * R)
    return pl.pallas_call(
        _pass_d_body,
        grid=(B, T),
        in_specs=[
            pl.BlockSpec((1, C_out), lambda b, t: (0, 0)),
            pl.BlockSpec((1, C_out), lambda b, t: (0, 0)),
            pl.BlockSpec((R, C_out), lambda b, t: (b * T + t, 0)),
            pl.BlockSpec((R, C_out), lambda b, t: (b * T + t, 0)),
        ],
        out_specs=pl.BlockSpec((GT, C_out), lambda b, t: (b * T + t, 0)),
        out_shape=jax.ShapeDtypeStruct((R_tot // K_NEIGHBORS, C_out), jnp.float32),
        compiler_params=pltpu.CompilerParams(
            dimension_semantics=("parallel", "parallel")),
    )(a2, c2, y2_rows, h_rows)


def _pe_consts(C_out):
    fd = C_out // 6
    freq = BETA / np.power(ALPHA, np.arange(fd, dtype=np.float64) / fd)
    sf = np.zeros((8, C_out), np.float32)
    phase = np.zeros((1, C_out), np.float32)
    for c in range(C_out):
        d = c // (2 * fd)
        t = c % (2 * fd)
        f = t if t < fd else t - fd
        sf[d, c] = freq[f]
        phase[0, c] = 0.0 if t < fd else np.pi / 2.0
    return jnp.asarray(sf), jnp.asarray(phase)


def _bn_affine(stats, gamma, beta, count):
    st = jnp.sum(stats.reshape(-1, 8, stats.shape[1]), axis=0)
    s, ss = st[0], st[1]
    mean = s / count
    var = ss / count - mean * mean
    a = gamma / jnp.sqrt(var + 1e-5)
    c = beta - mean * a
    return a[None, :], c[None, :]


def kernel(xyz, x, params):
    B, N = xyz.shape[0], xyz.shape[1]
    K = K_NEIGHBORS

    feat = jnp.einsum('oc,bcn->bon', params['w0'], x)
    m = jnp.mean(feat, axis=(0, 2), keepdims=True)
    v = jnp.var(feat, axis=(0, 2), keepdims=True)
    feat = (feat - m) / jnp.sqrt(v + 1e-5)
    feat = jax.nn.relu(feat * params['g0'][None, :, None]
                       + params['b0'][None, :, None])

    cur_xyz = xyz
    cur_rows = jnp.transpose(feat, (0, 2, 1))
    idx_key = jax.random.key(42)

    for i in range(4):
        Ni = cur_xyz.shape[1]
        G, C_out = GROUP_NUMS[i], OUT_DIMS[i]
        C_in = C_out // 2
        hd = C_out // 2

        fps_idx = jax.random.randint(
            jax.random.fold_in(idx_key, i), (B, G), 0, Ni)
        lc_xyz = _index_points(cur_xyz, fps_idx)

        dist = _square_distance(lc_xyz, cur_xyz)

        xyzw = jnp.pad(cur_xyz, ((0, 0), (0, 0), (0, 5)))
        knn_idx, st = _topk_select(dist, xyzw)

        ssum = jnp.sum(st, axis=(0, 1))
        sum_diff = ssum[0] + ssum[1] + ssum[2] - K * jnp.sum(lc_xyz)
        m3 = float(3 * B * G * K)
        var = (ssum[3] - sum_diff * sum_diff / m3) / (m3 - 1.0)
        invstd = jnp.full((1, 8), 1.0 / (jnp.sqrt(var) + 1e-5), jnp.float32)

        Dp = (C_in + 3 + 15) // 16 * 16
        tbl = jnp.concatenate([cur_rows, cur_xyz], axis=-1)
        tbl = tbl.reshape(B * Ni, C_in + 3)
        tbl = jnp.pad(tbl, ((0, 0), (0, Dp - C_in - 3)))
        base = jnp.arange(B, dtype=jnp.int32) * Ni
        gidx = knn_idx + base[:, None, None]
        knn_rows = _sc_gather(tbl, gidx.reshape(-1))
        lidx = (fps_idx + base[:, None]).reshape(-1)
        lc_rows = tbl[lidx]

        sf, phase = _pe_consts(C_out)
        w1t = params['w1_%d' % i].T
        w2t = params['w2_%d' % i].T

        y1_rows, h_rows, st1 = _pass_b(
            knn_rows, lc_rows, C_in, invstd, w1t, sf, phase)

        cnt = float(B * G * K)
        a1, c1 = _bn_affine(st1, params['g1_%d' % i], params['be1_%d' % i], cnt)
        y2_rows, st2 = _pass_c(y1_rows, w2t, a1, c1)
        a2, c2 = _bn_affine(st2, params['g2_%d' % i], params['be2_%d' % i], cnt)
        new_rows = _pass_d(y2_rows, h_rows, a2, c2)

        cur_rows = new_rows.reshape(B, G, C_out)
        cur_xyz = lc_xyz

    return cur_xyz, jnp.transpose(cur_rows, (0, 2, 1))


# speedup vs baseline: 2.8955x; 1.0011x over previous
import functools

import jax
import jax.numpy as jnp
import numpy as np
from jax.experimental import pallas as pl
from jax.experimental.pallas import tpu as pltpu
from jax.experimental.pallas import tpu_sc as plsc

EMBED_DIM = 36
OUT_DIMS = [72, 144, 288, 576]
GROUP_NUMS = [1024, 512, 256, 128]
K_NEIGHBORS = 40
ALPHA = 1000.0
BETA = 100.0

GT = 32


def _index_points(points, idx):
    B = points.shape[0]
    bidx = jnp.arange(B).reshape((B,) + (1,) * (idx.ndim - 1))
    return points[bidx, idx]


def _square_distance(src, dst):
    dist = -2.0 * jnp.matmul(src, jnp.transpose(dst, (0, 2, 1)))
    dist = dist + jnp.sum(src ** 2, -1)[:, :, None]
    dist = dist + jnp.sum(dst ** 2, -1)[:, None, :]
    return dist


_SC_CHUNK = 128


def _sc_gather(table, idx):
    V, Dp = table.shape
    M = idx.shape[0]
    NC, NS = 2, 16
    NW = NC * NS
    mpw = M // NW
    nch = mpw // _SC_CHUNK
    assert mpw % _SC_CHUNK == 0 and nch % 2 == 0

    mesh = plsc.VectorSubcoreMesh(core_axis_name="c", subcore_axis_name="s")

    @functools.partial(
        pl.kernel, mesh=mesh,
        out_type=jax.ShapeDtypeStruct((M, Dp), jnp.float32),
        scratch_types=[
            pltpu.VMEM((mpw,), jnp.int32),
            pltpu.VMEM((2, _SC_CHUNK, Dp), jnp.float32),
            pltpu.SemaphoreType.DMA,
        ],
        compiler_params=pltpu.CompilerParams(use_tc_tiling_on_sc=False),
    )
    def gather_kernel(table_hbm, idx_hbm, out_hbm, idx_v, rows_v, sem):
        wid = jax.lax.axis_index("s") * NC + jax.lax.axis_index("c")
        base0 = wid * mpw
        pltpu.sync_copy(idx_hbm.at[pl.ds(base0, mpw)], idx_v)
        pltpu.async_copy(
            table_hbm.at[idx_v.at[pl.ds(0, _SC_CHUNK)]], rows_v.at[0], sem)

        def body(j2, carry):
            for bf in range(2):
                j = 2 * j2 + bf
                pltpu.make_async_copy(
                    out_hbm.at[pl.ds(base0, _SC_CHUNK)],
                    rows_v.at[bf], sem).wait()

                @pl.when(j + 1 < nch)
                def _():
                    pltpu.async_copy(
                        table_hbm.at[idx_v.at[pl.ds((j + 1) * _SC_CHUNK,
                                                    _SC_CHUNK)]],
                        rows_v.at[1 - bf], sem)

                pltpu.sync_copy(
                    rows_v.at[bf],
                    out_hbm.at[pl.ds(base0 + j * _SC_CHUNK, _SC_CHUNK)])
            return carry

        jax.lax.fori_loop(0, nch // 2, body, 0)

    return gather_kernel(table, idx)


GTK = 128


def _topk_body(xyzw_ref, d_ref, idx_ref, st_ref):
    Gt, N = d_ref.shape
    K = K_NEIGHBORS
    d = d_ref[...]
    iota = jax.lax.broadcasted_iota(jnp.int32, (Gt, N), 1).astype(jnp.float32)
    kiota = jax.lax.broadcasted_iota(jnp.int32, (Gt, K), 1).astype(jnp.float32)
    idxs = jnp.zeros((Gt, K), jnp.float32)
    big = jnp.float32(1e9)
    for k in range(K):
        m = jnp.min(d, axis=1, keepdims=True)
        idx = jnp.min(jnp.where(d == m, iota, big), axis=1, keepdims=True)
        d = jnp.where(iota == idx, jnp.float32(jnp.inf), d)
        idxs = jnp.where(kiota == jnp.float32(k), idx, idxs)
    idx_ref[...] = idxs.astype(jnp.int32)
    W = jnp.isinf(d).astype(jnp.float32)
    sd = jnp.sum(W * d_ref[...], axis=1, keepdims=True)
    p = jax.lax.dot_general(
        W, xyzw_ref[0], (((1,), (0,)), ((), ())),
        precision=jax.lax.Precision.HIGHEST,
        preferred_element_type=jnp.float32)
    row = jnp.sum(p, axis=0, keepdims=True)
    sdt = jnp.sum(sd, axis=0, keepdims=True)
    slot3 = (jax.lax.broadcasted_iota(jnp.int32, (1, 8), 1) == 3
             ).astype(jnp.float32)
    st_ref[...] = (row + sdt * slot3).reshape(1, 1, 8)


def _topk_select(dist, xyzw):
    B, G, N = dist.shape
    T = G // GTK
    idx, st = pl.pallas_call(
        _topk_body,
        grid=(B, T),
        in_specs=[
            pl.BlockSpec((1, N, 8), lambda b, t: (b, 0, 0)),
            pl.BlockSpec((GTK, N), lambda b, t: (b * T + t, 0)),
        ],
        out_specs=[
            pl.BlockSpec((GTK, K_NEIGHBORS), lambda b, t: (b * T + t, 0)),
            pl.BlockSpec((1, 1, 8), lambda b, t: (b * T + t, 0, 0)),
        ],
        out_shape=[
            jax.ShapeDtypeStruct((B * G, K_NEIGHBORS), jnp.int32),
            jax.ShapeDtypeStruct((B * T, 1, 8), jnp.float32),
        ],
        compiler_params=pltpu.CompilerParams(
            dimension_semantics=("parallel", "parallel")),
    )(xyzw, dist.reshape(B * G, N))
    return idx.reshape(B, G, K_NEIGHBORS), st


def _pass_b_body(T, C_in, Dp, rows_ref, lc_ref, inv_ref, w1t_ref, sf_ref,
                 ph_ref, y1_ref, h_ref, st_ref):
    b, t = pl.program_id(0), pl.program_id(1)
    K = K_NEIGHBORS

    rows = rows_ref[...]
    lcb = jnp.broadcast_to(
        lc_ref[...][:, None, :], (GT, K, Dp)).reshape(GT * K, Dp)
    xn = (rows[:, C_in:C_in + 8] - lcb[:, C_in:C_in + 8]) * inv_ref[0:1, 0:1]
    arg = jax.lax.dot_general(
        xn, sf_ref[...], (((1,), (0,)), ((), ())),
        precision=jax.lax.Precision.HIGHEST,
        preferred_element_type=jnp.float32)
    pe = jnp.sin(arg + ph_ref[...])
    h = jnp.concatenate(
        [rows[:, 0:C_in], lcb[:, 0:C_in]], axis=1) + pe
    h_ref[...] = h

    y1 = jax.lax.dot_general(
        h, w1t_ref[...], (((1,), (0,)), ((), ())),
        precision=jax.lax.Precision.HIGHEST,
        preferred_element_type=jnp.float32)
    y1_ref[...] = y1

    @pl.when(t == 0)
    def _():
        st_ref[...] = jnp.zeros_like(st_ref)

    st_ref[0:1, :] += jnp.sum(y1, axis=0, keepdims=True)
    st_ref[1:2, :] += jnp.sum(y1 * y1, axis=0, keepdims=True)


def _pass_b(knn_rows, lc_rows, C_in, invstd, w1t, sf, phase):
    R_tot, Dp = knn_rows.shape
    C_out, hd = w1t.shape
    B = 8
    T = R_tot // (B * GT * K_NEIGHBORS)
    R = GT * K_NEIGHBORS
    return pl.pallas_call(
        functools.partial(_pass_b_body, T, C_in, Dp),
        grid=(B, T),
        in_specs=[
            pl.BlockSpec((R, Dp), lambda b, t: (b * T + t, 0)),
            pl.BlockSpec((GT, Dp), lambda b, t: (b * T + t, 0)),
            pl.BlockSpec((1, 8), lambda b, t: (0, 0)),
            pl.BlockSpec((C_out, hd), lambda b, t: (0, 0)),
            pl.BlockSpec((8, C_out), lambda b, t: (0, 0)),
            pl.BlockSpec((1, C_out), lambda b, t: (0, 0)),
        ],
        out_specs=[
            pl.BlockSpec((R, hd), lambda b, t: (b * T + t, 0)),
            pl.BlockSpec((R, C_out), lambda b, t: (b * T + t, 0)),
            pl.BlockSpec((8, hd), lambda b, t: (b, 0)),
        ],
        out_shape=[
            jax.ShapeDtypeStruct((R_tot, hd), jnp.float32),
            jax.ShapeDtypeStruct((R_tot, C_out), jnp.float32),
            jax.ShapeDtypeStruct((8 * B, hd), jnp.float32),
        ],
        compiler_params=pltpu.CompilerParams(
            dimension_semantics=("parallel", "arbitrary")),
    )(knn_rows, lc_rows, invstd, w1t, sf, phase)


def _pass_c_body(a1_ref, c1_ref, y1_ref, w2t_ref, y2_ref, st_ref):
    b, t = pl.program_id(0), pl.program_id(1)
    r1 = jnp.maximum(y1_ref[...] * a1_ref[...] + c1_ref[...], 0.0)
    y2 = jax.lax.dot_general(
        r1, w2t_ref[...], (((1,), (0,)), ((), ())),
        precision=jax.lax.Precision.HIGHEST,
        preferred_element_type=jnp.float32)
    y2_ref[...] = y2

    @pl.when(t == 0)
    def _():
        st_ref[...] = jnp.zeros_like(st_ref)

    st_ref[0:1, :] += jnp.sum(y2, axis=0, keepdims=True)
    st_ref[1:2, :] += jnp.sum(y2 * y2, axis=0, keepdims=True)


def _pass_c(y1_rows, w2t, a1, c1):
    R_tot, hd = y1_rows.shape
    C_out = w2t.shape[1]
    B = 8
    R = GT * K_NEIGHBORS
    T = R_tot // (B * R)
    return pl.pallas_call(
        _pass_c_body,
        grid=(B, T),
        in_specs=[
            pl.BlockSpec((1, hd), lambda b, t: (0, 0)),
            pl.BlockSpec((1, hd), lambda b, t: (0, 0)),
            pl.BlockSpec((R, hd), lambda b, t: (b * T + t, 0)),
            pl.BlockSpec((hd, C_out), lambda b, t: (0, 0)),
        ],
        out_specs=[
            pl.BlockSpec((R, C_out), lambda b, t: (b * T + t, 0)),
            pl.BlockSpec((8, C_out), lambda b, t: (b, 0)),
        ],
        out_shape=[
            jax.ShapeDtypeStruct((R_tot, C_out), jnp.float32),
            jax.ShapeDtypeStruct((8 * B, C_out), jnp.float32),
        ],
        compiler_params=pltpu.CompilerParams(
            dimension_semantics=("parallel", "arbitrary")),
    )(a1, c1, y1_rows, w2t)


def _pass_d_body(a2_ref, c2_ref, y2_ref, h_ref, o_ref):
    C_out = y2_ref.shape[1]
    hout = jnp.maximum(y2_ref[...] * a2_ref[...] + c2_ref[...] + h_ref[...], 0.0)
    hr = hout.reshape(GT, K_NEIGHBORS, C_out)
    o_ref[...] = jnp.max(hr, axis=1)


def _pass_d(y2_rows, h_rows, a2, c2):
    R_tot, C_out = y2_rows.shape
    B = 8
    R = GT * K_NEIGHBORS
    T = R_tot // (B * R)
    return pl.pallas_call(
        _pass_d_body,
        grid=(B, T),
        in_specs=[
            pl.BlockSpec((1, C_out), lambda b, t: (0, 0)),
            pl.BlockSpec((1, C_out), lambda b, t: (0, 0)),
            pl.BlockSpec((R, C_out), lambda b, t: (b * T + t, 0)),
            pl.BlockSpec((R, C_out), lambda b, t: (b * T + t, 0)),
        ],
        out_specs=pl.BlockSpec((GT, C_out), lambda b, t: (b * T + t, 0)),
        out_shape=jax.ShapeDtypeStruct((R_tot // K_NEIGHBORS, C_out), jnp.float32),
        compiler_params=pltpu.CompilerParams(
            dimension_semantics=("parallel", "parallel")),
    )(a2, c2, y2_rows, h_rows)


def _pe_consts(C_out):
    fd = C_out // 6
    freq = BETA / np.power(ALPHA, np.arange(fd, dtype=np.float64) / fd)
    sf = np.zeros((8, C_out), np.float32)
    phase = np.zeros((1, C_out), np.float32)
    for c in range(C_out):
        d = c // (2 * fd)
        t = c % (2 * fd)
        f = t if t < fd else t - fd
        sf[d, c] = freq[f]
        phase[0, c] = 0.0 if t < fd else np.pi / 2.0
    return jnp.asarray(sf), jnp.asarray(phase)


def _bn_affine(stats, gamma, beta, count):
    st = jnp.sum(stats.reshape(-1, 8, stats.shape[1]), axis=0)
    s, ss = st[0], st[1]
    mean = s / count
    var = ss / count - mean * mean
    a = gamma / jnp.sqrt(var + 1e-5)
    c = beta - mean * a
    return a[None, :], c[None, :]


def kernel(xyz, x, params):
    B, N = xyz.shape[0], xyz.shape[1]
    K = K_NEIGHBORS

    feat = jnp.einsum('oc,bcn->bon', params['w0'], x)
    m = jnp.mean(feat, axis=(0, 2), keepdims=True)
    v = jnp.var(feat, axis=(0, 2), keepdims=True)
    feat = (feat - m) / jnp.sqrt(v + 1e-5)
    feat = jax.nn.relu(feat * params['g0'][None, :, None]
                       + params['b0'][None, :, None])

    cur_xyz = xyz
    cur_rows = jnp.transpose(feat, (0, 2, 1))
    idx_key = jax.random.key(42)

    for i in range(4):
        Ni = cur_xyz.shape[1]
        G, C_out = GROUP_NUMS[i], OUT_DIMS[i]
        C_in = C_out // 2
        hd = C_out // 2

        fps_idx = jax.random.randint(
            jax.random.fold_in(idx_key, i), (B, G), 0, Ni)
        lc_xyz = _index_points(cur_xyz, fps_idx)

        dist = _square_distance(lc_xyz, cur_xyz)

        xyzw = jnp.pad(cur_xyz, ((0, 0), (0, 0), (0, 5)))
        knn_idx, st = _topk_select(dist, xyzw)

        ssum = jnp.sum(st, axis=(0, 1))
        sum_diff = ssum[0] + ssum[1] + ssum[2] - K * jnp.sum(lc_xyz)
        m3 = float(3 * B * G * K)
        var = (ssum[3] - sum_diff * sum_diff / m3) / (m3 - 1.0)
        invstd = jnp.full((1, 8), 1.0 / (jnp.sqrt(var) + 1e-5), jnp.float32)

        Dp = (C_in + 3 + 15) // 16 * 16
        tbl = jnp.concatenate([cur_rows, cur_xyz], axis=-1)
        tbl = tbl.reshape(B * Ni, C_in + 3)
        tbl = jnp.pad(tbl, ((0, 0), (0, Dp - C_in - 3)))
        base = jnp.arange(B, dtype=jnp.int32) * Ni
        gidx = knn_idx + base[:, None, None]
        knn_rows = _sc_gather(tbl, gidx.reshape(-1))
        lidx = (fps_idx + base[:, None]).reshape(-1)
        lc_rows = tbl[lidx]

        sf, phase = _pe_consts(C_out)
        w1t = params['w1_%d' % i].T
        w2t = params['w2_%d' % i].T

        y1_rows, h_rows, st1 = _pass_b(
            knn_rows, lc_rows, C_in, invstd, w1t, sf, phase)

        cnt = float(B * G * K)
        a1, c1 = _bn_affine(st1, params['g1_%d' % i], params['be1_%d' % i], cnt)
        y2_rows, st2 = _pass_c(y1_rows, w2t, a1, c1)
        a2, c2 = _bn_affine(st2, params['g2_%d' % i], params['be2_%d' % i], cnt)
        new_rows = _pass_d(y2_rows, h_rows, a2, c2)

        cur_rows = new_rows.reshape(B, G, C_out)
        cur_xyz = lc_xyz

    return cur_xyz, jnp.transpose(cur_rows, (0, 2, 1))


# speedup vs baseline: 3.0172x; 1.0420x over previous
import functools

import jax
import jax.numpy as jnp
import numpy as np
from jax.experimental import pallas as pl
from jax.experimental.pallas import tpu as pltpu
from jax.experimental.pallas import tpu_sc as plsc

EMBED_DIM = 36
OUT_DIMS = [72, 144, 288, 576]
GROUP_NUMS = [1024, 512, 256, 128]
K_NEIGHBORS = 40
ALPHA = 1000.0
BETA = 100.0

GT = 64


def _index_points(points, idx):
    B = points.shape[0]
    bidx = jnp.arange(B).reshape((B,) + (1,) * (idx.ndim - 1))
    return points[bidx, idx]


def _square_distance(src, dst):
    dist = -2.0 * jnp.matmul(src, jnp.transpose(dst, (0, 2, 1)))
    dist = dist + jnp.sum(src ** 2, -1)[:, :, None]
    dist = dist + jnp.sum(dst ** 2, -1)[:, None, :]
    return dist


_SC_CHUNK = 128


def _sc_gather(table, idx):
    V, Dp = table.shape
    M = idx.shape[0]
    NC, NS = 2, 16
    NW = NC * NS
    mpw = M // NW
    nch = mpw // _SC_CHUNK
    assert mpw % _SC_CHUNK == 0 and nch % 2 == 0

    mesh = plsc.VectorSubcoreMesh(core_axis_name="c", subcore_axis_name="s")

    @functools.partial(
        pl.kernel, mesh=mesh,
        out_type=jax.ShapeDtypeStruct((M, Dp), jnp.float32),
        scratch_types=[
            pltpu.VMEM((mpw,), jnp.int32),
            pltpu.VMEM((2, _SC_CHUNK, Dp), jnp.float32),
            pltpu.SemaphoreType.DMA,
        ],
        compiler_params=pltpu.CompilerParams(use_tc_tiling_on_sc=False),
    )
    def gather_kernel(table_hbm, idx_hbm, out_hbm, idx_v, rows_v, sem):
        wid = jax.lax.axis_index("s") * NC + jax.lax.axis_index("c")
        base0 = wid * mpw
        pltpu.sync_copy(idx_hbm.at[pl.ds(base0, mpw)], idx_v)
        pltpu.async_copy(
            table_hbm.at[idx_v.at[pl.ds(0, _SC_CHUNK)]], rows_v.at[0], sem)

        def body(j2, carry):
            for bf in range(2):
                j = 2 * j2 + bf
                pltpu.make_async_copy(
                    out_hbm.at[pl.ds(base0, _SC_CHUNK)],
                    rows_v.at[bf], sem).wait()

                @pl.when(j + 1 < nch)
                def _():
                    pltpu.async_copy(
                        table_hbm.at[idx_v.at[pl.ds((j + 1) * _SC_CHUNK,
                                                    _SC_CHUNK)]],
                        rows_v.at[1 - bf], sem)

                pltpu.sync_copy(
                    rows_v.at[bf],
                    out_hbm.at[pl.ds(base0 + j * _SC_CHUNK, _SC_CHUNK)])
            return carry

        jax.lax.fori_loop(0, nch // 2, body, 0)

    return gather_kernel(table, idx)


GTK = 256


def _topk_body(xyzw_ref, d_ref, idx_ref, st_ref):
    Gt, N = d_ref.shape
    K = K_NEIGHBORS
    d = d_ref[...]
    iota = jax.lax.broadcasted_iota(jnp.int32, (Gt, N), 1).astype(jnp.float32)
    kiota = jax.lax.broadcasted_iota(jnp.int32, (Gt, K), 1).astype(jnp.float32)
    idxs = jnp.zeros((Gt, K), jnp.float32)
    big = jnp.float32(1e9)
    for k in range(K):
        m = jnp.min(d, axis=1, keepdims=True)
        idx = jnp.min(jnp.where(d == m, iota, big), axis=1, keepdims=True)
        d = jnp.where(iota == idx, jnp.float32(jnp.inf), d)
        idxs = jnp.where(kiota == jnp.float32(k), idx, idxs)
    idx_ref[...] = idxs.astype(jnp.int32)
    W = jnp.isinf(d).astype(jnp.float32)
    sd = jnp.sum(W * d_ref[...], axis=1, keepdims=True)
    p = jax.lax.dot_general(
        W, xyzw_ref[0], (((1,), (0,)), ((), ())),
        precision=jax.lax.Precision.HIGHEST,
        preferred_element_type=jnp.float32)
    row = jnp.sum(p, axis=0, keepdims=True)
    sdt = jnp.sum(sd, axis=0, keepdims=True)
    slot3 = (jax.lax.broadcasted_iota(jnp.int32, (1, 8), 1) == 3
             ).astype(jnp.float32)
    st_ref[...] = (row + sdt * slot3).reshape(1, 1, 8)


def _topk_select(dist, xyzw):
    B, G, N = dist.shape
    gt = min(GTK, G)
    T = G // gt
    idx, st = pl.pallas_call(
        _topk_body,
        grid=(B, T),
        in_specs=[
            pl.BlockSpec((1, N, 8), lambda b, t: (b, 0, 0)),
            pl.BlockSpec((gt, N), lambda b, t: (b * T + t, 0)),
        ],
        out_specs=[
            pl.BlockSpec((gt, K_NEIGHBORS), lambda b, t: (b * T + t, 0)),
            pl.BlockSpec((1, 1, 8), lambda b, t: (b * T + t, 0, 0)),
        ],
        out_shape=[
            jax.ShapeDtypeStruct((B * G, K_NEIGHBORS), jnp.int32),
            jax.ShapeDtypeStruct((B * T, 1, 8), jnp.float32),
        ],
        compiler_params=pltpu.CompilerParams(
            dimension_semantics=("parallel", "parallel")),
    )(xyzw, dist.reshape(B * G, N))
    return idx.reshape(B, G, K_NEIGHBORS), st


def _pass_b_body(T, C_in, Dp, rows_ref, lc_ref, inv_ref, w1t_ref, sf_ref,
                 ph_ref, y1_ref, h_ref, st_ref):
    b, t = pl.program_id(0), pl.program_id(1)
    K = K_NEIGHBORS

    rows = rows_ref[...]
    lcb = jnp.broadcast_to(
        lc_ref[...][:, None, :], (GT, K, Dp)).reshape(GT * K, Dp)
    xn = (rows[:, C_in:C_in + 8] - lcb[:, C_in:C_in + 8]) * inv_ref[0:1, 0:1]
    arg = jax.lax.dot_general(
        xn, sf_ref[...], (((1,), (0,)), ((), ())),
        precision=jax.lax.Precision.HIGHEST,
        preferred_element_type=jnp.float32)
    pe = jnp.sin(arg + ph_ref[...])
    h = jnp.concatenate(
        [rows[:, 0:C_in], lcb[:, 0:C_in]], axis=1) + pe
    h_ref[...] = h

    y1 = jax.lax.dot_general(
        h, w1t_ref[...], (((1,), (0,)), ((), ())),
        precision=jax.lax.Precision.HIGHEST,
        preferred_element_type=jnp.float32)
    y1_ref[...] = y1

    @pl.when(t == 0)
    def _():
        st_ref[...] = jnp.zeros_like(st_ref)

    st_ref[0:1, :] += jnp.sum(y1, axis=0, keepdims=True)
    st_ref[1:2, :] += jnp.sum(y1 * y1, axis=0, keepdims=True)


def _pass_b(knn_rows, lc_rows, C_in, invstd, w1t, sf, phase):
    R_tot, Dp = knn_rows.shape
    C_out, hd = w1t.shape
    B = 8
    T = R_tot // (B * GT * K_NEIGHBORS)
    R = GT * K_NEIGHBORS
    return pl.pallas_call(
        functools.partial(_pass_b_body, T, C_in, Dp),
        grid=(B, T),
        in_specs=[
            pl.BlockSpec((R, Dp), lambda b, t: (b * T + t, 0)),
            pl.BlockSpec((GT, Dp), lambda b, t: (b * T + t, 0)),
            pl.BlockSpec((1, 8), lambda b, t: (0, 0)),
            pl.BlockSpec((C_out, hd), lambda b, t: (0, 0)),
            pl.BlockSpec((8, C_out), lambda b, t: (0, 0)),
            pl.BlockSpec((1, C_out), lambda b, t: (0, 0)),
        ],
        out_specs=[
            pl.BlockSpec((R, hd), lambda b, t: (b * T + t, 0)),
            pl.BlockSpec((R, C_out), lambda b, t: (b * T + t, 0)),
            pl.BlockSpec((8, hd), lambda b, t: (b, 0)),
        ],
        out_shape=[
            jax.ShapeDtypeStruct((R_tot, hd), jnp.float32),
            jax.ShapeDtypeStruct((R_tot, C_out), jnp.float32),
            jax.ShapeDtypeStruct((8 * B, hd), jnp.float32),
        ],
        compiler_params=pltpu.CompilerParams(
            dimension_semantics=("parallel", "arbitrary")),
    )(knn_rows, lc_rows, invstd, w1t, sf, phase)


def _pass_c_body(a1_ref, c1_ref, y1_ref, w2t_ref, y2_ref, st_ref):
    b, t = pl.program_id(0), pl.program_id(1)
    r1 = jnp.maximum(y1_ref[...] * a1_ref[...] + c1_ref[...], 0.0)
    y2 = jax.lax.dot_general(
        r1, w2t_ref[...], (((1,), (0,)), ((), ())),
        precision=jax.lax.Precision.HIGHEST,
        preferred_element_type=jnp.float32)
    y2_ref[...] = y2

    @pl.when(t == 0)
    def _():
        st_ref[...] = jnp.zeros_like(st_ref)

    st_ref[0:1, :] += jnp.sum(y2, axis=0, keepdims=True)
    st_ref[1:2, :] += jnp.sum(y2 * y2, axis=0, keepdims=True)


def _pass_c(y1_rows, w2t, a1, c1):
    R_tot, hd = y1_rows.shape
    C_out = w2t.shape[1]
    B = 8
    R = GT * K_NEIGHBORS
    T = R_tot // (B * R)
    return pl.pallas_call(
        _pass_c_body,
        grid=(B, T),
        in_specs=[
            pl.BlockSpec((1, hd), lambda b, t: (0, 0)),
            pl.BlockSpec((1, hd), lambda b, t: (0, 0)),
            pl.BlockSpec((R, hd), lambda b, t: (b * T + t, 0)),
            pl.BlockSpec((hd, C_out), lambda b, t: (0, 0)),
        ],
        out_specs=[
            pl.BlockSpec((R, C_out), lambda b, t: (b * T + t, 0)),
            pl.BlockSpec((8, C_out), lambda b, t: (b, 0)),
        ],
        out_shape=[
            jax.ShapeDtypeStruct((R_tot, C_out), jnp.float32),
            jax.ShapeDtypeStruct((8 * B, C_out), jnp.float32),
        ],
        compiler_params=pltpu.CompilerParams(
            dimension_semantics=("parallel", "arbitrary")),
    )(a1, c1, y1_rows, w2t)


def _pass_d_body(a2_ref, c2_ref, y2_ref, h_ref, o_ref):
    C_out = y2_ref.shape[1]
    hout = jnp.maximum(y2_ref[...] * a2_ref[...] + c2_ref[...] + h_ref[...], 0.0)
    hr = hout.reshape(GT, K_NEIGHBORS, C_out)
    o_ref[...] = jnp.max(hr, axis=1)


def _pass_d(y2_rows, h_rows, a2, c2):
    R_tot, C_out = y2_rows.shape
    B = 8
    R = GT * K_NEIGHBORS
    T = R_tot // (B * R)
    return pl.pallas_call(
        _pass_d_body,
        grid=(B, T),
        in_specs=[
            pl.BlockSpec((1, C_out), lambda b, t: (0, 0)),
            pl.BlockSpec((1, C_out), lambda b, t: (0, 0)),
            pl.BlockSpec((R, C_out), lambda b, t: (b * T + t, 0)),
            pl.BlockSpec((R, C_out), lambda b, t: (b * T + t, 0)),
        ],
        out_specs=pl.BlockSpec((GT, C_out), lambda b, t: (b * T + t, 0)),
        out_shape=jax.ShapeDtypeStruct((R_tot // K_NEIGHBORS, C_out), jnp.float32),
        compiler_params=pltpu.CompilerParams(
            dimension_semantics=("parallel", "parallel")),
    )(a2, c2, y2_rows, h_rows)


def _pe_consts(C_out):
    fd = C_out // 6
    freq = BETA / np.power(ALPHA, np.arange(fd, dtype=np.float64) / fd)
    sf = np.zeros((8, C_out), np.float32)
    phase = np.zeros((1, C_out), np.float32)
    for c in range(C_out):
        d = c // (2 * fd)
        t = c % (2 * fd)
        f = t if t < fd else t - fd
        sf[d, c] = freq[f]
        phase[0, c] = 0.0 if t < fd else np.pi / 2.0
    return jnp.asarray(sf), jnp.asarray(phase)


def _bn_affine(stats, gamma, beta, count):
    st = jnp.sum(stats.reshape(-1, 8, stats.shape[1]), axis=0)
    s, ss = st[0], st[1]
    mean = s / count
    var = ss / count - mean * mean
    a = gamma / jnp.sqrt(var + 1e-5)
    c = beta - mean * a
    return a[None, :], c[None, :]


def kernel(xyz, x, params):
    B, N = xyz.shape[0], xyz.shape[1]
    K = K_NEIGHBORS

    feat = jnp.einsum('oc,bcn->bon', params['w0'], x)
    m = jnp.mean(feat, axis=(0, 2), keepdims=True)
    v = jnp.var(feat, axis=(0, 2), keepdims=True)
    feat = (feat - m) / jnp.sqrt(v + 1e-5)
    feat = jax.nn.relu(feat * params['g0'][None, :, None]
                       + params['b0'][None, :, None])

    cur_xyz = xyz
    cur_rows = jnp.transpose(feat, (0, 2, 1))
    idx_key = jax.random.key(42)

    for i in range(4):
        Ni = cur_xyz.shape[1]
        G, C_out = GROUP_NUMS[i], OUT_DIMS[i]
        C_in = C_out // 2
        hd = C_out // 2

        fps_idx = jax.random.randint(
            jax.random.fold_in(idx_key, i), (B, G), 0, Ni)
        lc_xyz = _index_points(cur_xyz, fps_idx)

        dist = _square_distance(lc_xyz, cur_xyz)

        xyzw = jnp.pad(cur_xyz, ((0, 0), (0, 0), (0, 5)))
        knn_idx, st = _topk_select(dist, xyzw)

        ssum = jnp.sum(st, axis=(0, 1))
        sum_diff = ssum[0] + ssum[1] + ssum[2] - K * jnp.sum(lc_xyz)
        m3 = float(3 * B * G * K)
        var = (ssum[3] - sum_diff * sum_diff / m3) / (m3 - 1.0)
        invstd = jnp.full((1, 8), 1.0 / (jnp.sqrt(var) + 1e-5), jnp.float32)

        Dp = (C_in + 3 + 15) // 16 * 16
        tbl = jnp.concatenate([cur_rows, cur_xyz], axis=-1)
        tbl = tbl.reshape(B * Ni, C_in + 3)
        tbl = jnp.pad(tbl, ((0, 0), (0, Dp - C_in - 3)))
        base = jnp.arange(B, dtype=jnp.int32) * Ni
        gidx = knn_idx + base[:, None, None]
        knn_rows = _sc_gather(tbl, gidx.reshape(-1))
        lidx = (fps_idx + base[:, None]).reshape(-1)
        lc_rows = tbl[lidx]

        sf, phase = _pe_consts(C_out)
        w1t = params['w1_%d' % i].T
        w2t = params['w2_%d' % i].T

        y1_rows, h_rows, st1 = _pass_b(
            knn_rows, lc_rows, C_in, invstd, w1t, sf, phase)

        cnt = float(B * G * K)
        a1, c1 = _bn_affine(st1, params['g1_%d' % i], params['be1_%d' % i], cnt)
        y2_rows, st2 = _pass_c(y1_rows, w2t, a1, c1)
        a2, c2 = _bn_affine(st2, params['g2_%d' % i], params['be2_%d' % i], cnt)
        new_rows = _pass_d(y2_rows, h_rows, a2, c2)

        cur_rows = new_rows.reshape(B, G, C_out)
        cur_xyz = lc_xyz

    return cur_xyz, jnp.transpose(cur_rows, (0, 2, 1))
